# Initial kernel scaffold; baseline (speedup 1.0000x reference)
import jax
import jax.numpy as jnp
from jax.experimental import pallas as pl


def kernel(feat0, feat1, feat2, edge_index, Wf0, bf0, Wf1, bf1, Wf2, bf2, W2, b2, semantic_weight):
    raise NotImplementedError("write your pallas kernel here")


# speedup vs baseline: 3.3244x; 3.3244x over previous
import functools

import jax
import jax.numpy as jnp
from jax import lax
from jax.experimental import pallas as pl
from jax.experimental.pallas import tpu as pltpu
from jax.experimental.pallas import tpu_sc as plsc

N_NODES = 10000
N_EDGES = 320000
HID = 128

NC = 2
NS = 16
NW = NC * NS

N_PAD = 10112
ROWS_PER_TILE = N_PAD // NS

CH = 128
E_TILE = 10112
NCH = E_TILE // CH
N_EDGES_P = NW * E_TILE
JUNK_ROW = 10016

DEG_W = 128
D_TILE = N_EDGES_P // NS
DCH = 128
NDCH = D_TILE // DCH


@functools.cache
def _mesh():
    return plsc.VectorSubcoreMesh(
        core_axis_name="c", subcore_axis_name="s", num_cores=NC, num_subcores=NS
    )


_WB_CHUNKS = [(0, 128), (128, 128), (256, 128), (384, 128), (512, 120)]


def _deg_body(ids_hbm, ones_hbm, zeros_hbm, out_hbm, idx_v, ones_v, zbuf_v, hist_sh):
    c = lax.axis_index("c")
    s = lax.axis_index("s")
    pltpu.sync_copy(zeros_hbm, zbuf_v)
    row0 = s * ROWS_PER_TILE
    for off, sz in _WB_CHUNKS:
        pltpu.sync_copy(zbuf_v.at[pl.ds(0, sz)], hist_sh.at[pl.ds(row0 + off, sz)])
    pltpu.sync_copy(ones_hbm, ones_v)
    plsc.subcore_barrier()
    my_ids = ids_hbm.at[c]

    def body(i, carry):
        pltpu.sync_copy(my_ids.at[pl.ds(s * D_TILE + i * DCH, DCH)], idx_v)
        pltpu.sync_copy(ones_v, hist_sh.at[idx_v], add=True)
        return carry

    lax.fori_loop(0, NDCH, body, 0)
    plsc.subcore_barrier()
    for off, sz in _WB_CHUNKS:
        pltpu.sync_copy(hist_sh.at[pl.ds(row0 + off, sz)], zbuf_v.at[pl.ds(0, sz)])
        pltpu.sync_copy(zbuf_v.at[pl.ds(0, sz)], out_hbm.at[c, pl.ds(row0 + off, sz)])


def _msgpass_body(h_hbm, src_hbm, dst_hbm, zeros_hbm, out_hbm,
                  src_v, dst_v, msg_v, agg_sh, sem):
    c = lax.axis_index("c")
    s = lax.axis_index("s")
    pltpu.sync_copy(zeros_hbm, msg_v)
    row0 = s * ROWS_PER_TILE
    for off, sz in _WB_CHUNKS:
        pltpu.sync_copy(msg_v.at[pl.ds(0, sz)], agg_sh.at[pl.ds(row0 + off, sz)])
    plsc.subcore_barrier()
    base = (c * NS + s) * E_TILE

    def body(i, carry):
        pltpu.sync_copy(src_hbm.at[pl.ds(base + i * CH, CH)], src_v)
        pltpu.sync_copy(dst_hbm.at[pl.ds(base + i * CH, CH)], dst_v)
        pltpu.async_copy(h_hbm.at[src_v], msg_v, sem).wait()
        pltpu.sync_copy(msg_v, agg_sh.at[dst_v], add=True)
        return carry

    lax.fori_loop(0, NCH, body, 0)
    plsc.subcore_barrier()
    for off, sz in _WB_CHUNKS:
        pltpu.sync_copy(agg_sh.at[pl.ds(row0 + off, sz)], msg_v.at[pl.ds(0, sz)])
        pltpu.sync_copy(msg_v.at[pl.ds(0, sz)], out_hbm.at[c, pl.ds(row0 + off, sz)])


@functools.cache
def _deg_kernel():
    return pl.kernel(
        _deg_body,
        out_type=jax.ShapeDtypeStruct((NC, N_PAD, DEG_W), jnp.float32),
        mesh=_mesh(),
        scratch_types=[
            pltpu.VMEM((DCH,), jnp.int32),
            pltpu.VMEM((DCH, DEG_W), jnp.float32),
            pltpu.VMEM((DCH, DEG_W), jnp.float32),
            pltpu.VMEM_SHARED((N_PAD, DEG_W), jnp.float32),
        ],
    )


@functools.cache
def _msgpass_kernel():
    return pl.kernel(
        _msgpass_body,
        out_type=jax.ShapeDtypeStruct((NC, N_PAD, HID), jnp.float32),
        mesh=_mesh(),
        scratch_types=[
            pltpu.VMEM((CH,), jnp.int32),
            pltpu.VMEM((CH,), jnp.int32),
            pltpu.VMEM((CH, HID), jnp.float32),
            pltpu.VMEM_SHARED((N_PAD, HID), jnp.float32),
            pltpu.SemaphoreType.DMA,
        ],
    )


def _proj_body(f0_ref, f1_ref, f2_ref, w0_ref, b0_ref, w1_ref, b1_ref,
               w2_ref, b2_ref, hist_ref, hh_ref, ns_ref, nd_ref):
    h0 = jnp.dot(f0_ref[...], w0_ref[...], preferred_element_type=jnp.float32) + b0_ref[...]
    h1 = jnp.dot(f1_ref[...], w1_ref[...], preferred_element_type=jnp.float32) + b1_ref[...]
    h2 = jnp.dot(f2_ref[...], w2_ref[...], preferred_element_type=jnp.float32) + b2_ref[...]
    h = jnp.concatenate([h0, h1, h2], axis=0)
    out_deg = hist_ref[0, :N_NODES, 0]
    in_deg = hist_ref[1, :N_NODES, 0]
    ns = lax.rsqrt(jnp.maximum(out_deg, 1.0))
    nd = lax.rsqrt(jnp.maximum(in_deg, 1.0))
    hh_ref[...] = h * ns[:, None]
    ns_ref[...] = ns[:, None]
    nd_ref[...] = nd[:, None]


def _scale_body(p_ref, nd_ref, ns_ref, out_ref):
    a = p_ref[0, :N_NODES, :] + p_ref[1, :N_NODES, :]
    out_ref[...] = jnp.maximum(a * nd_ref[...], 0.0) * ns_ref[...]


def _final_body(q_ref, nd_ref, w2_ref, b2_ref, out_ref):
    a = q_ref[0, :N_NODES, :] + q_ref[1, :N_NODES, :]
    a = a * nd_ref[...]
    z = jnp.dot(a, w2_ref[...], preferred_element_type=jnp.float32) + b2_ref[...]
    out_ref[...] = jnp.maximum(z, 0.0)


def kernel(feat0, feat1, feat2, edge_index, Wf0, bf0, Wf1, bf1, Wf2, bf2,
           W2, b2, semantic_weight):
    src = edge_index[0].astype(jnp.int32)
    dst = edge_index[1].astype(jnp.int32)
    n_pad_edges = N_EDGES_P - N_EDGES
    src_mp = jnp.concatenate([src, jnp.zeros((n_pad_edges,), jnp.int32)])
    dst_mp = jnp.concatenate(
        [dst, jnp.full((n_pad_edges,), JUNK_ROW, jnp.int32)])
    junk = jnp.full((n_pad_edges,), JUNK_ROW, jnp.int32)
    ids = jnp.stack([jnp.concatenate([src, junk]), jnp.concatenate([dst, junk])])

    ones_deg = jnp.ones((DCH, DEG_W), jnp.float32)
    zeros_deg = jnp.zeros((DCH, DEG_W), jnp.float32)
    zeros_mp = jnp.zeros((CH, HID), jnp.float32)

    hist = _deg_kernel()(ids, ones_deg, zeros_deg)

    hh0, ns, nd = pl.pallas_call(
        _proj_body,
        out_shape=[
            jax.ShapeDtypeStruct((N_NODES, HID), jnp.float32),
            jax.ShapeDtypeStruct((N_NODES, 1), jnp.float32),
            jax.ShapeDtypeStruct((N_NODES, 1), jnp.float32),
        ],
    )(feat0, feat1, feat2, Wf0, bf0, Wf1, bf1, Wf2, bf2, hist)

    p0 = _msgpass_kernel()(hh0, src_mp, dst_mp, zeros_mp)

    hh1 = pl.pallas_call(
        _scale_body,
        out_shape=jax.ShapeDtypeStruct((N_NODES, HID), jnp.float32),
    )(p0, nd, ns)

    p1 = _msgpass_kernel()(hh1, src_mp, dst_mp, zeros_mp)

    out = pl.pallas_call(
        _final_body,
        out_shape=jax.ShapeDtypeStruct((N_NODES, HID), jnp.float32),
    )(p1, nd, W2, b2)

    return (out, semantic_weight)
